# baseline (speedup 1.0000x reference)
import jax
import jax.numpy as jnp
from jax.experimental import pallas as pl
from jax.experimental.pallas import tpu as pltpu
from jax.sharding import Mesh, PartitionSpec as P


def _lstm_kernel(x_ref, len_ref, wih_ref, whh_ref, b_ref, h0_ref,
                 out_ref, xt_ref, gbuf_ref, sems):
    T, Bt, E = xt_ref.shape
    H2 = whh_ref.shape[0]
    H = H2 // 2
    H4 = 4 * H
    CH = gbuf_ref.shape[2]
    NC = T // CH

    def wave_rows(ci):
        lo = ci * CH
        return list(range(lo, lo + CH)) + list(range(T - lo - CH, T - lo))

    for ci in range(NC // 2):
        for t in wave_rows(ci):
            pltpu.make_async_copy(
                x_ref.at[:, t], xt_ref.at[t], sems.at[ci]).start()

    def wait_wave(ci):
        for t in wave_rows(ci):
            pltpu.make_async_copy(
                xt_ref.at[t], xt_ref.at[t], sems.at[ci]).wait()

    w_f = jnp.concatenate(
        [wih_ref[:E, k * H2:k * H2 + H] for k in range(4)], axis=1)
    w_b = jnp.concatenate(
        [wih_ref[E:, k * H2 + H:(k + 1) * H2] for k in range(4)], axis=1)
    whf = jnp.concatenate(
        [whh_ref[:H, k * H2:k * H2 + H] for k in range(4)],
        axis=1).astype(jnp.bfloat16)
    whb = jnp.concatenate(
        [whh_ref[H:, k * H2 + H:(k + 1) * H2] for k in range(4)],
        axis=1).astype(jnp.bfloat16)
    b_f = jnp.concatenate(
        [b_ref[:, k * H2:k * H2 + H] for k in range(4)], axis=1)
    b_b = jnp.concatenate(
        [b_ref[:, k * H2 + H:(k + 1) * H2] for k in range(4)], axis=1)

    def proj_chunk(ci):
        if ci < NC // 2:
            wait_wave(ci)
        buf = ci % 2
        lo = ci * CH
        xf = xt_ref[lo:lo + CH].reshape(CH * Bt, E).astype(jnp.bfloat16)
        gbuf_ref[buf, 0] = (
            jnp.dot(xf, w_f, preferred_element_type=jnp.float32)
            + b_f).reshape(CH, Bt, H4)
        s = T - lo - CH
        xb = xt_ref[s:s + CH].reshape(CH * Bt, E).astype(jnp.bfloat16)
        gbuf_ref[buf, 1] = (
            jnp.dot(xb, w_b, preferred_element_type=jnp.float32)
            + b_b).reshape(CH, Bt, H4)

    lens = len_ref[...]

    def cell(gates, h, c, m):
        sg_if = 0.5 * jnp.tanh(0.5 * gates[:, :2 * H]) + 0.5
        i_g = sg_if[:, :H]
        f_g = sg_if[:, H:]
        o_g = 0.5 * jnp.tanh(0.5 * gates[:, 3 * H:]) + 0.5
        g_g = jnp.tanh(gates[:, 2 * H:3 * H])
        c_new = f_g * c + i_g * g_g
        h_new = o_g * jnp.tanh(c_new)
        return jnp.where(m, h_new, h), jnp.where(m, c_new, c)

    hf = h0_ref[0]
    hb = h0_ref[1]
    cf = jnp.zeros((Bt, H), jnp.float32)
    cb = jnp.zeros((Bt, H), jnp.float32)

    proj_chunk(0)
    for ci in range(NC):
        if ci + 1 < NC:
            proj_chunk(ci + 1)
        buf = ci % 2
        lo = ci * CH
        for k in range(CH):
            t = lo + k
            gf = gbuf_ref[buf, 0, k] + jnp.dot(
                hf.astype(jnp.bfloat16), whf,
                preferred_element_type=jnp.float32)
            gb = gbuf_ref[buf, 1, CH - 1 - k] + jnp.dot(
                hb.astype(jnp.bfloat16), whb,
                preferred_element_type=jnp.float32)
            hf, cf = cell(gf, hf, cf, t < lens)
            hb, cb = cell(gb, hb, cb, (T - 1 - t) < lens)

    out_ref[...] = jnp.concatenate([hf, hb], axis=1)


def _encode_tile(x_bte, lens, h0, w_ih, w_hh, b):
    Bt, T, E = x_bte.shape
    H2 = w_hh.shape[0]
    H = H2 // 2
    CH = 8 if T % 16 == 0 else 1
    NC = T // CH
    grid_spec = pltpu.PrefetchScalarGridSpec(
        num_scalar_prefetch=0,
        grid=(1,),
        in_specs=[
            pl.BlockSpec(memory_space=pl.ANY),
            pl.BlockSpec((Bt, 1), lambda i: (0, 0)),
            pl.BlockSpec((2 * E, 8 * H), lambda i: (0, 0)),
            pl.BlockSpec((H2, 8 * H), lambda i: (0, 0)),
            pl.BlockSpec((1, 8 * H), lambda i: (0, 0)),
            pl.BlockSpec((2, Bt, H), lambda i: (0, 0, 0)),
        ],
        out_specs=pl.BlockSpec((Bt, H2), lambda i: (0, 0)),
        scratch_shapes=[
            pltpu.VMEM((T, Bt, E), jnp.float32),
            pltpu.VMEM((2, 2, CH, Bt, 4 * H), jnp.float32),
            pltpu.SemaphoreType.DMA((NC // 2,)),
        ],
    )
    return pl.pallas_call(
        _lstm_kernel,
        out_shape=jax.ShapeDtypeStruct((Bt, H2), jnp.float32),
        grid_spec=grid_spec,
    )(x_bte, lens, w_ih, w_hh, b, h0)


@jax.jit
def kernel(x_bte, lengths, h0, w_ih, w_hh, b, dir):
    del dir
    B = x_bte.shape[0]
    lens = lengths.astype(jnp.int32).reshape(B, 1)
    wih_bf = w_ih.astype(jnp.bfloat16)
    whh_f = w_hh.astype(jnp.float32)
    b_f32 = b.astype(jnp.float32)
    h0_f = h0.astype(jnp.float32)

    devs = jax.devices()
    n_sh = 2 if (len(devs) >= 2 and B % 2 == 0) else 1
    if n_sh == 1:
        return _encode_tile(x_bte, lens, h0_f, wih_bf, whh_f, b_f32)

    mesh = Mesh(devs[:n_sh], ("b",))
    fn = jax.shard_map(
        _encode_tile, mesh=mesh,
        in_specs=(P("b", None, None), P("b", None), P(None, "b", None),
                  P(None, None), P(None, None), P(None, None)),
        out_specs=P("b", None),
        check_vma=False)
    return fn(x_bte, lens, h0_f, wih_bf, whh_f, b_f32)


# speedup vs baseline: 8.6467x; 8.6467x over previous
import jax
import jax.numpy as jnp
from jax.experimental import pallas as pl
from jax.experimental.pallas import tpu as pltpu


def _lstm_kernel(x_ref, len_ref, wih_ref, whh_ref, b_ref, h0_ref,
                 out_ref, xt_ref, gbuf_ref, sems):
    T, Bt, E = xt_ref.shape
    H2 = whh_ref.shape[0]
    H = H2 // 2
    H4 = 4 * H
    CH = gbuf_ref.shape[2]
    NC = T // CH

    b0 = pl.program_id(0) * Bt

    def wave_rows(ci):
        lo = ci * CH
        return list(range(lo, lo + CH)) + list(range(T - lo - CH, T - lo))

    for ci in range(NC // 2):
        for t in wave_rows(ci):
            pltpu.make_async_copy(
                x_ref.at[pl.ds(b0, Bt), t], xt_ref.at[t], sems.at[ci]).start()

    def wait_wave(ci):
        for t in wave_rows(ci):
            pltpu.make_async_copy(
                xt_ref.at[t], xt_ref.at[t], sems.at[ci]).wait()

    sc = [0.5, 0.5, 1.0, 0.5]

    def compact(ref, rows, col0):
        return jnp.concatenate(
            [ref[rows, k * H2 + col0:k * H2 + col0 + H] *
             jnp.array(sc[k], ref.dtype) for k in range(4)], axis=1)

    w_f = compact(wih_ref, slice(0, E), 0)
    w_b = compact(wih_ref, slice(E, 2 * E), H)
    whf = compact(whh_ref, slice(0, H), 0).astype(jnp.bfloat16)
    whb = compact(whh_ref, slice(H, H2), H).astype(jnp.bfloat16)
    b_f = compact(b_ref, slice(0, 1), 0)
    b_b = compact(b_ref, slice(0, 1), H)

    def proj_chunk(ci):
        if ci < NC // 2:
            wait_wave(ci)
        buf = ci % 2
        lo = ci * CH
        xf = xt_ref[lo:lo + CH].reshape(CH * Bt, E).astype(jnp.bfloat16)
        gbuf_ref[buf, 0] = (
            jnp.dot(xf, w_f, preferred_element_type=jnp.float32)
            + b_f).reshape(CH, Bt, H4)
        s = T - lo - CH
        xb = xt_ref[s:s + CH].reshape(CH * Bt, E).astype(jnp.bfloat16)
        gbuf_ref[buf, 1] = (
            jnp.dot(xb, w_b, preferred_element_type=jnp.float32)
            + b_b).reshape(CH, Bt, H4)

    lens = len_ref[...]

    def cell(gates, h, c, m, mask_c):
        sg_if = 0.5 * jnp.tanh(gates[:, :2 * H]) + 0.5
        i_g = sg_if[:, :H]
        f_g = sg_if[:, H:]
        o_g = 0.5 * jnp.tanh(gates[:, 3 * H:]) + 0.5
        g_g = jnp.tanh(gates[:, 2 * H:3 * H])
        c_new = f_g * c + i_g * g_g
        h_new = o_g * jnp.tanh(c_new)
        c_out = jnp.where(m, c_new, c) if mask_c else c_new
        return jnp.where(m, h_new, h), c_out

    hf = h0_ref[0]
    hb = h0_ref[1]
    cf = jnp.zeros((Bt, H), jnp.float32)
    cb = jnp.zeros((Bt, H), jnp.float32)

    proj_chunk(0)
    for ci in range(NC):
        if ci + 1 < NC:
            proj_chunk(ci + 1)
        buf = ci % 2
        lo = ci * CH
        for k in range(CH):
            t = lo + k
            gf = gbuf_ref[buf, 0, k] + jnp.dot(
                hf.astype(jnp.bfloat16), whf,
                preferred_element_type=jnp.float32)
            gb = gbuf_ref[buf, 1, CH - 1 - k] + jnp.dot(
                hb.astype(jnp.bfloat16), whb,
                preferred_element_type=jnp.float32)
            hf, cf = cell(gf, hf, cf, t < lens, False)
            hb, cb = cell(gb, hb, cb, (T - 1 - t) < lens, True)

    out_ref[...] = jnp.concatenate([hf, hb], axis=1)


@jax.jit
def kernel(x_bte, lengths, h0, w_ih, w_hh, b, dir):
    del dir
    B, T, E = x_bte.shape
    H2 = w_hh.shape[0]
    H = H2 // 2

    lens = lengths.astype(jnp.int32).reshape(B, 1)

    Bt = B // 2 if (B // 2) % 8 == 0 else B
    nb = B // Bt
    CH = 8 if T % 16 == 0 else 1
    NC = T // CH
    grid_spec = pltpu.PrefetchScalarGridSpec(
        num_scalar_prefetch=0,
        grid=(nb,),
        in_specs=[
            pl.BlockSpec(memory_space=pl.ANY),
            pl.BlockSpec((Bt, 1), lambda i: (i, 0)),
            pl.BlockSpec((2 * E, 8 * H), lambda i: (0, 0)),
            pl.BlockSpec((H2, 8 * H), lambda i: (0, 0)),
            pl.BlockSpec((1, 8 * H), lambda i: (0, 0)),
            pl.BlockSpec((2, Bt, H), lambda i: (0, i, 0)),
        ],
        out_specs=pl.BlockSpec((Bt, H2), lambda i: (i, 0)),
        scratch_shapes=[
            pltpu.VMEM((T, Bt, E), jnp.float32),
            pltpu.VMEM((2, 2, CH, Bt, 4 * H), jnp.float32),
            pltpu.SemaphoreType.DMA((NC // 2,)),
        ],
    )
    out = pl.pallas_call(
        _lstm_kernel,
        out_shape=jax.ShapeDtypeStruct((B, H2), jnp.float32),
        grid_spec=grid_spec,
        compiler_params=pltpu.CompilerParams(
            dimension_semantics=("parallel",)),
    )(x_bte, lens, w_ih.astype(jnp.bfloat16), w_hh.astype(jnp.float32),
      b.astype(jnp.float32), h0.astype(jnp.float32))
    return out
